# baseline (speedup 1.0000x reference)
import functools

import jax
import jax.numpy as jnp
from jax import lax
from jax.experimental import pallas as pl
from jax.experimental.pallas import tpu as pltpu
from jax.experimental.pallas import tpu_sc as plsc

N = 25000
E = 400000
D = 128
H = 128
OUT = 64
NS = 16
NC = 2
WL = 64
EPAD = 409600
NWIN = EPAD // WL
WPT = NWIN // NS
CH = 40
NCHUNK = WPT // CH

HALF0 = 12504
HALF1 = N - HALF0
SINK = 12544
NPAD = 12672
ZROWS = NPAD // NS
OROWS = 776

_mesh = plsc.VectorSubcoreMesh(
    core_axis_name="c", subcore_axis_name="s", num_cores=NC, num_subcores=NS)


def _fill_slot(ref, slot, nrows, value):
  v = jnp.full((16,), value, jnp.float32)

  def body(i, carry):
    ref[slot, i // 8, pl.ds((i % 8) * 16, 16)] = v
    return carry

  lax.fori_loop(0, nrows * 8, body, 0)


def _fill2d(ref, nrows, value):
  v = jnp.full((16,), value, jnp.float32)

  def body(i, carry):
    ref[i // 8, pl.ds((i % 8) * 16, 16)] = v
    return carry

  lax.fori_loop(0, nrows * 8, body, 0)


def _zero_acc_slice(zsrc, acc, sid):
  base = sid * ZROWS
  for j in range(ZROWS // WL):
    pltpu.sync_copy(zsrc, acc.at[pl.ds(base + j * WL, WL)])
  tail = ZROWS % WL
  if tail:
    pltpu.sync_copy(zsrc.at[pl.ds(0, tail)],
                    acc.at[pl.ds(base + (ZROWS // WL) * WL, tail)])


def _localize(didx, base, bound):
  kpr = WL // 16

  def body(i, carry):
    w = i // kpr
    k = i % kpr
    d = didx[w, pl.ds(k * 16, 16)]
    loc = d - base
    oob = (loc < 0) | (loc >= bound)
    snk = SINK + (d & 127)
    didx[w, pl.ds(k * 16, 16)] = jnp.where(oob, snk, loc)
    return carry

  lax.fori_loop(0, CH * kpr, body, 0)


def _writeout(acc, out, sid, cbase, tail):
  pltpu.sync_copy(acc.at[pl.ds(sid * OROWS, OROWS)],
                  out.at[pl.ds(cbase + sid * OROWS, OROWS)])

  @pl.when(sid == 0)
  def _():
    pltpu.sync_copy(acc.at[pl.ds(NS * OROWS, tail)],
                    out.at[pl.ds(cbase + NS * OROWS, tail)])


def _seg_body(tableA, sA, dA, tableB, sB, dB, outA, outB,
              sidx, didx, rows, acc, gsem0, gsem1):
  cid = lax.axis_index("c")
  sid = lax.axis_index("s")

  for table, s2d, d2d, out in ((tableA, sA, dA, outA), (tableB, sB, dB, outB)):
    _seg_phase(table, s2d, d2d, out, sidx, didx, rows, acc, gsem0, gsem1,
               cid, sid)


def _seg_phase(table, s2d, d2d, out, sidx, didx, rows, acc, gsem0, gsem1,
               cid, sid):
  _fill_slot(rows, 0, WL, 0.0)
  _zero_acc_slice(rows.at[0], acc, sid)
  plsc.subcore_barrier()

  def run(base, bound):
    def chunk(c, carry):
      cb = sid * WPT + c * CH
      pltpu.sync_copy(s2d.at[pl.ds(cb, CH)], sidx)
      pltpu.sync_copy(d2d.at[pl.ds(cb, CH)], didx)
      _localize(didx, base, bound)
      pltpu.async_copy(table.at[sidx.at[0]], rows.at[0], gsem0)

      def pair(p, carry2):
        w0 = 2 * p
        w1 = w0 + 1
        pltpu.async_copy(table.at[sidx.at[w1]], rows.at[1], gsem1)
        pltpu.make_async_copy(table.at[sidx.at[w0]], rows.at[0], gsem0).wait()
        pltpu.sync_copy(rows.at[0], acc.at[didx.at[w0]], add=True)

        @pl.when(p + 1 < CH // 2)
        def _():
          pltpu.async_copy(table.at[sidx.at[w0 + 2]], rows.at[0], gsem0)

        pltpu.make_async_copy(table.at[sidx.at[w1]], rows.at[1], gsem1).wait()
        pltpu.sync_copy(rows.at[1], acc.at[didx.at[w1]], add=True)
        return carry2

      lax.fori_loop(0, CH // 2, pair, 0)
      return carry

    lax.fori_loop(0, NCHUNK, chunk, 0)

  @pl.when(cid == 0)
  def _():
    run(0, HALF0)

  @pl.when(cid == 1)
  def _():
    run(HALF0, HALF1)

  plsc.subcore_barrier()

  @pl.when(cid == 0)
  def _():
    _writeout(acc, out, sid, 0, HALF0 - NS * OROWS)

  @pl.when(cid == 1)
  def _():
    _writeout(acc, out, sid, HALF0, HALF1 - NS * OROWS)

  plsc.subcore_barrier()


_seg128 = functools.partial(
    pl.kernel,
    _seg_body,
    out_type=(jax.ShapeDtypeStruct((N, H), jnp.float32),
              jax.ShapeDtypeStruct((N, H), jnp.float32)),
    mesh=_mesh,
    scratch_types=[
        pltpu.VMEM((CH, WL), jnp.int32),
        pltpu.VMEM((CH, WL), jnp.int32),
        pltpu.VMEM((2, WL, H), jnp.float32),
        pltpu.VMEM_SHARED((NPAD, H), jnp.float32),
        pltpu.SemaphoreType.DMA,
        pltpu.SemaphoreType.DMA,
    ],
)()


def _cnt_body(dA, dB, outA, outB, didx, ones, acc, ssem):
  cid = lax.axis_index("c")
  sid = lax.axis_index("s")

  for d2d, out in ((dA, outA), (dB, outB)):
    _cnt_phase(d2d, out, didx, ones, acc, ssem, cid, sid)


def _cnt_phase(d2d, out, didx, ones, acc, ssem, cid, sid):
  _fill2d(ones, WL, 0.0)
  _zero_acc_slice(ones, acc, sid)
  _fill2d(ones, WL, 1.0)
  plsc.subcore_barrier()

  def run(base, bound):
    def chunk(c, carry):
      cb = sid * WPT + c * CH
      pltpu.sync_copy(d2d.at[pl.ds(cb, CH)], didx)
      _localize(didx, base, bound)
      pltpu.async_copy(ones, acc.at[didx.at[0]], ssem, add=True)

      def win(w, carry2):
        @pl.when(w + 1 < CH)
        def _():
          pltpu.async_copy(ones, acc.at[didx.at[w + 1]], ssem, add=True)

        pltpu.make_async_copy(ones, acc.at[didx.at[w]], ssem).wait()
        return carry2

      lax.fori_loop(0, CH, win, 0)
      return carry

    lax.fori_loop(0, NCHUNK, chunk, 0)

  @pl.when(cid == 0)
  def _():
    run(0, HALF0)

  @pl.when(cid == 1)
  def _():
    run(HALF0, HALF1)

  plsc.subcore_barrier()

  @pl.when(cid == 0)
  def _():
    _writeout(acc, out, sid, 0, HALF0 - NS * OROWS)

  @pl.when(cid == 1)
  def _():
    _writeout(acc, out, sid, HALF0, HALF1 - NS * OROWS)

  plsc.subcore_barrier()


_cnt128 = functools.partial(
    pl.kernel,
    _cnt_body,
    out_type=(jax.ShapeDtypeStruct((N, H), jnp.float32),
              jax.ShapeDtypeStruct((N, H), jnp.float32)),
    mesh=_mesh,
    scratch_types=[
        pltpu.VMEM((CH, WL), jnp.int32),
        pltpu.VMEM((WL, H), jnp.float32),
        pltpu.VMEM_SHARED((NPAD, H), jnp.float32),
        pltpu.SemaphoreType.DMA,
    ],
)()


BN = 1000
NB = N // BN


def _proj_body(x, wt, b, h):
  y = jnp.dot(x[...], wt[...], preferred_element_type=jnp.float32) + b[...]
  h[...] = jnp.maximum(y, 0.0)


def _proj(x, wt, b):
  return pl.pallas_call(
      _proj_body,
      grid=(NB,),
      in_specs=[
          pl.BlockSpec((BN, D), lambda i: (i, 0)),
          pl.BlockSpec((D, H), lambda i: (0, 0)),
          pl.BlockSpec((1, H), lambda i: (0, 0)),
      ],
      out_specs=pl.BlockSpec((BN, H), lambda i: (i, 0)),
      out_shape=jax.ShapeDtypeStruct((N, H), jnp.float32),
  )(x, wt, b)


def _conv1_body(ssum, cnt, h, wl, b, wr, wr2, s_out, r_out):
  icnt = 1.0 / jnp.maximum(cnt[...][:, 0:1], 1.0)
  y = (jnp.dot(ssum[...], wl[...], preferred_element_type=jnp.float32) * icnt +
       b[...] +
       jnp.dot(h[...], wr[...], preferred_element_type=jnp.float32))
  s = jnp.maximum(y, 0.0)
  s_out[...] = s
  r_out[...] = jnp.dot(s, wr2[...], preferred_element_type=jnp.float32)


def _conv1(ssum, cnt, h, wl, b, wr, wr2):
  return pl.pallas_call(
      _conv1_body,
      grid=(NB,),
      in_specs=[
          pl.BlockSpec((BN, H), lambda i: (i, 0)),
          pl.BlockSpec((BN, H), lambda i: (i, 0)),
          pl.BlockSpec((BN, H), lambda i: (i, 0)),
          pl.BlockSpec((H, H), lambda i: (0, 0)),
          pl.BlockSpec((1, H), lambda i: (0, 0)),
          pl.BlockSpec((H, H), lambda i: (0, 0)),
          pl.BlockSpec((H, OUT), lambda i: (0, 0)),
      ],
      out_specs=[
          pl.BlockSpec((BN, H), lambda i: (i, 0)),
          pl.BlockSpec((BN, OUT), lambda i: (i, 0)),
      ],
      out_shape=[
          jax.ShapeDtypeStruct((N, H), jnp.float32),
          jax.ShapeDtypeStruct((N, OUT), jnp.float32),
      ],
  )(ssum, cnt, h, wl, b, wr, wr2)


def _final_body(ssum, cnt, r, wl, b, out):
  icnt = 1.0 / jnp.maximum(cnt[...][:, 0:1], 1.0)
  out[...] = (jnp.dot(ssum[...], wl[...], preferred_element_type=jnp.float32)
              * icnt + b[...] + r[...])


def _final(ssum, cnt, r, wl, b):
  return pl.pallas_call(
      _final_body,
      grid=(NB,),
      in_specs=[
          pl.BlockSpec((BN, H), lambda i: (i, 0)),
          pl.BlockSpec((BN, H), lambda i: (i, 0)),
          pl.BlockSpec((BN, OUT), lambda i: (i, 0)),
          pl.BlockSpec((H, OUT), lambda i: (0, 0)),
          pl.BlockSpec((1, OUT), lambda i: (0, 0)),
      ],
      out_specs=pl.BlockSpec((BN, OUT), lambda i: (i, 0)),
      out_shape=jax.ShapeDtypeStruct((N, OUT), jnp.float32),
  )(ssum, cnt, r, wl, b)


def _pad_edges(src, dst):
  npad = EPAD - E
  pad = jnp.arange(npad, dtype=jnp.int32) % 128
  s2d = jnp.concatenate([src.astype(jnp.int32), pad]).reshape(NWIN, WL)
  d2d = jnp.concatenate([dst.astype(jnp.int32), N + pad]).reshape(NWIN, WL)
  return s2d, d2d


@jax.jit
def kernel(x_occupation, x_skill, edge_index_requires, edge_index_rev_requires,
           Wp_occ, bp_occ, Wp_skill, bp_skill,
           Wl1_rs, bl1_rs, Wr1_rs, Wl1_so, bl1_so, Wr1_so,
           Wl2_rs, bl2_rs, Wr2_rs, Wl2_so, bl2_so, Wr2_so):
  req_s2d, req_d2d = _pad_edges(edge_index_requires[0], edge_index_requires[1])
  rev_s2d, rev_d2d = _pad_edges(edge_index_rev_requires[0],
                                edge_index_rev_requires[1])

  h_occ = _proj(x_occupation, Wp_occ.T, bp_occ[None, :])
  h_sk = _proj(x_skill, Wp_skill.T, bp_skill[None, :])

  cnt_req, cnt_rev = _cnt128(req_d2d, rev_d2d)

  agg_s1, agg_o1 = _seg128(h_occ, req_s2d, req_d2d, h_sk, rev_s2d, rev_d2d)

  o1, r_o = _conv1(agg_o1, cnt_rev, h_occ, Wl1_so.T, bl1_so[None, :],
                   Wr1_so.T, Wr2_so.T)
  s1, r_s = _conv1(agg_s1, cnt_req, h_sk, Wl1_rs.T, bl1_rs[None, :],
                   Wr1_rs.T, Wr2_rs.T)

  agg_s2, agg_o2 = _seg128(o1, req_s2d, req_d2d, s1, rev_s2d, rev_d2d)

  out_occ = _final(agg_o2, cnt_rev, r_o, Wl2_so.T, bl2_so[None, :])
  out_sk = _final(agg_s2, cnt_req, r_s, Wl2_rs.T, bl2_rs[None, :])
  return (out_occ, out_sk)


# speedup vs baseline: 1.0542x; 1.0542x over previous
import functools

import jax
import jax.numpy as jnp
from jax import lax
from jax.experimental import pallas as pl
from jax.experimental.pallas import tpu as pltpu
from jax.experimental.pallas import tpu_sc as plsc

N = 25000
E = 400000
D = 128
H = 128
OUT = 64
NS = 16
NC = 2
WL = 64
EPAD = 409600
NWIN = EPAD // WL
WPT = NWIN // NS
CH = 40
NCHUNK = WPT // CH

HALF0 = 12504
HALF1 = N - HALF0
SINK = 12544
NPAD = 12672
ZROWS = NPAD // NS
OROWS = 776

_mesh = plsc.VectorSubcoreMesh(
    core_axis_name="c", subcore_axis_name="s", num_cores=NC, num_subcores=NS)


def _fill_slot(ref, slot, nrows, value):
  v = jnp.full((16,), value, jnp.float32)

  def body(i, carry):
    ref[slot, i // 8, pl.ds((i % 8) * 16, 16)] = v
    return carry

  lax.fori_loop(0, nrows * 8, body, 0)


def _fill2d(ref, nrows, value):
  v = jnp.full((16,), value, jnp.float32)

  def body(i, carry):
    ref[i // 8, pl.ds((i % 8) * 16, 16)] = v
    return carry

  lax.fori_loop(0, nrows * 8, body, 0)


def _zero_acc_slice(zsrc, acc, sid):
  base = sid * ZROWS
  for j in range(ZROWS // WL):
    pltpu.sync_copy(zsrc, acc.at[pl.ds(base + j * WL, WL)])
  tail = ZROWS % WL
  if tail:
    pltpu.sync_copy(zsrc.at[pl.ds(0, tail)],
                    acc.at[pl.ds(base + (ZROWS // WL) * WL, tail)])


def _localize(didx, base, bound):
  kpr = WL // 16

  def body(i, carry):
    w = i // kpr
    k = i % kpr
    d = didx[w, pl.ds(k * 16, 16)]
    loc = d - base
    oob = (loc < 0) | (loc >= bound)
    snk = SINK + (d & 127)
    didx[w, pl.ds(k * 16, 16)] = jnp.where(oob, snk, loc)
    return carry

  lax.fori_loop(0, CH * kpr, body, 0)


def _writeout(acc, out, sid, cbase, tail):
  pltpu.sync_copy(acc.at[pl.ds(sid * OROWS, OROWS)],
                  out.at[pl.ds(cbase + sid * OROWS, OROWS)])

  @pl.when(sid == 0)
  def _():
    pltpu.sync_copy(acc.at[pl.ds(NS * OROWS, tail)],
                    out.at[pl.ds(cbase + NS * OROWS, tail)])


def _seg_body(table, s2d, d2d, out, sidx, didx, rows, acc, gsem0, gsem1):
  cid = lax.axis_index("c")
  sid = lax.axis_index("s")

  _fill_slot(rows, 0, WL, 0.0)
  _zero_acc_slice(rows.at[0], acc, sid)
  plsc.subcore_barrier()

  def run(base, bound):
    def chunk(c, carry):
      cb = sid * WPT + c * CH
      pltpu.sync_copy(s2d.at[pl.ds(cb, CH)], sidx)
      pltpu.sync_copy(d2d.at[pl.ds(cb, CH)], didx)
      _localize(didx, base, bound)
      pltpu.async_copy(table.at[sidx.at[0]], rows.at[0], gsem0)

      def pair(p, carry2):
        w0 = 2 * p
        w1 = w0 + 1
        pltpu.async_copy(table.at[sidx.at[w1]], rows.at[1], gsem1)
        pltpu.make_async_copy(table.at[sidx.at[w0]], rows.at[0], gsem0).wait()
        pltpu.sync_copy(rows.at[0], acc.at[didx.at[w0]], add=True)

        @pl.when(p + 1 < CH // 2)
        def _():
          pltpu.async_copy(table.at[sidx.at[w0 + 2]], rows.at[0], gsem0)

        pltpu.make_async_copy(table.at[sidx.at[w1]], rows.at[1], gsem1).wait()
        pltpu.sync_copy(rows.at[1], acc.at[didx.at[w1]], add=True)
        return carry2

      lax.fori_loop(0, CH // 2, pair, 0)
      return carry

    lax.fori_loop(0, NCHUNK, chunk, 0)

  @pl.when(cid == 0)
  def _():
    run(0, HALF0)

  @pl.when(cid == 1)
  def _():
    run(HALF0, HALF1)

  plsc.subcore_barrier()

  @pl.when(cid == 0)
  def _():
    _writeout(acc, out, sid, 0, HALF0 - NS * OROWS)

  @pl.when(cid == 1)
  def _():
    _writeout(acc, out, sid, HALF0, HALF1 - NS * OROWS)


_seg128 = functools.partial(
    pl.kernel,
    _seg_body,
    out_type=jax.ShapeDtypeStruct((N, H), jnp.float32),
    mesh=_mesh,
    scratch_types=[
        pltpu.VMEM((CH, WL), jnp.int32),
        pltpu.VMEM((CH, WL), jnp.int32),
        pltpu.VMEM((2, WL, H), jnp.float32),
        pltpu.VMEM_SHARED((NPAD, H), jnp.float32),
        pltpu.SemaphoreType.DMA,
        pltpu.SemaphoreType.DMA,
    ],
)()


def _cnt_body(d2d, out, didx, ones, acc, ssem):
  cid = lax.axis_index("c")
  sid = lax.axis_index("s")

  _fill2d(ones, WL, 0.0)
  _zero_acc_slice(ones, acc, sid)
  _fill2d(ones, WL, 1.0)
  plsc.subcore_barrier()

  def run(base, bound):
    def chunk(c, carry):
      cb = sid * WPT + c * CH
      pltpu.sync_copy(d2d.at[pl.ds(cb, CH)], didx)
      _localize(didx, base, bound)
      pltpu.async_copy(ones, acc.at[didx.at[0]], ssem, add=True)

      def win(w, carry2):
        @pl.when(w + 1 < CH)
        def _():
          pltpu.async_copy(ones, acc.at[didx.at[w + 1]], ssem, add=True)

        pltpu.make_async_copy(ones, acc.at[didx.at[w]], ssem).wait()
        return carry2

      lax.fori_loop(0, CH, win, 0)
      return carry

    lax.fori_loop(0, NCHUNK, chunk, 0)

  @pl.when(cid == 0)
  def _():
    run(0, HALF0)

  @pl.when(cid == 1)
  def _():
    run(HALF0, HALF1)

  plsc.subcore_barrier()

  @pl.when(cid == 0)
  def _():
    _writeout(acc, out, sid, 0, HALF0 - NS * OROWS)

  @pl.when(cid == 1)
  def _():
    _writeout(acc, out, sid, HALF0, HALF1 - NS * OROWS)


_cnt128 = functools.partial(
    pl.kernel,
    _cnt_body,
    out_type=jax.ShapeDtypeStruct((N, H), jnp.float32),
    mesh=_mesh,
    scratch_types=[
        pltpu.VMEM((CH, WL), jnp.int32),
        pltpu.VMEM((WL, H), jnp.float32),
        pltpu.VMEM_SHARED((NPAD, H), jnp.float32),
        pltpu.SemaphoreType.DMA,
    ],
)()


BN = 1000
NB = N // BN


def _proj_body(x, wt, b, h):
  y = jnp.dot(x[...], wt[...], preferred_element_type=jnp.float32) + b[...]
  h[...] = jnp.maximum(y, 0.0)


def _proj(x, wt, b):
  return pl.pallas_call(
      _proj_body,
      grid=(NB,),
      in_specs=[
          pl.BlockSpec((BN, D), lambda i: (i, 0)),
          pl.BlockSpec((D, H), lambda i: (0, 0)),
          pl.BlockSpec((1, H), lambda i: (0, 0)),
      ],
      out_specs=pl.BlockSpec((BN, H), lambda i: (i, 0)),
      out_shape=jax.ShapeDtypeStruct((N, H), jnp.float32),
  )(x, wt, b)


def _conv1_body(ssum, cnt, h, wl, b, wr, wr2, s_out, r_out):
  icnt = 1.0 / jnp.maximum(cnt[...][:, 0:1], 1.0)
  y = (jnp.dot(ssum[...], wl[...], preferred_element_type=jnp.float32) * icnt +
       b[...] +
       jnp.dot(h[...], wr[...], preferred_element_type=jnp.float32))
  s = jnp.maximum(y, 0.0)
  s_out[...] = s
  r_out[...] = jnp.dot(s, wr2[...], preferred_element_type=jnp.float32)


def _conv1(ssum, cnt, h, wl, b, wr, wr2):
  return pl.pallas_call(
      _conv1_body,
      grid=(NB,),
      in_specs=[
          pl.BlockSpec((BN, H), lambda i: (i, 0)),
          pl.BlockSpec((BN, H), lambda i: (i, 0)),
          pl.BlockSpec((BN, H), lambda i: (i, 0)),
          pl.BlockSpec((H, H), lambda i: (0, 0)),
          pl.BlockSpec((1, H), lambda i: (0, 0)),
          pl.BlockSpec((H, H), lambda i: (0, 0)),
          pl.BlockSpec((H, OUT), lambda i: (0, 0)),
      ],
      out_specs=[
          pl.BlockSpec((BN, H), lambda i: (i, 0)),
          pl.BlockSpec((BN, OUT), lambda i: (i, 0)),
      ],
      out_shape=[
          jax.ShapeDtypeStruct((N, H), jnp.float32),
          jax.ShapeDtypeStruct((N, OUT), jnp.float32),
      ],
  )(ssum, cnt, h, wl, b, wr, wr2)


def _final_body(ssum, cnt, r, wl, b, out):
  icnt = 1.0 / jnp.maximum(cnt[...][:, 0:1], 1.0)
  out[...] = (jnp.dot(ssum[...], wl[...], preferred_element_type=jnp.float32)
              * icnt + b[...] + r[...])


def _final(ssum, cnt, r, wl, b):
  return pl.pallas_call(
      _final_body,
      grid=(NB,),
      in_specs=[
          pl.BlockSpec((BN, H), lambda i: (i, 0)),
          pl.BlockSpec((BN, H), lambda i: (i, 0)),
          pl.BlockSpec((BN, OUT), lambda i: (i, 0)),
          pl.BlockSpec((H, OUT), lambda i: (0, 0)),
          pl.BlockSpec((1, OUT), lambda i: (0, 0)),
      ],
      out_specs=pl.BlockSpec((BN, OUT), lambda i: (i, 0)),
      out_shape=jax.ShapeDtypeStruct((N, OUT), jnp.float32),
  )(ssum, cnt, r, wl, b)


def _pad_edges(src, dst):
  npad = EPAD - E
  pad = jnp.arange(npad, dtype=jnp.int32) % 128
  s2d = jnp.concatenate([src.astype(jnp.int32), pad]).reshape(NWIN, WL)
  d2d = jnp.concatenate([dst.astype(jnp.int32), N + pad]).reshape(NWIN, WL)
  return s2d, d2d


@jax.jit
def kernel(x_occupation, x_skill, edge_index_requires, edge_index_rev_requires,
           Wp_occ, bp_occ, Wp_skill, bp_skill,
           Wl1_rs, bl1_rs, Wr1_rs, Wl1_so, bl1_so, Wr1_so,
           Wl2_rs, bl2_rs, Wr2_rs, Wl2_so, bl2_so, Wr2_so):
  req_s2d, req_d2d = _pad_edges(edge_index_requires[0], edge_index_requires[1])
  rev_s2d, rev_d2d = _pad_edges(edge_index_rev_requires[0],
                                edge_index_rev_requires[1])

  h_occ = _proj(x_occupation, Wp_occ.T, bp_occ[None, :])
  h_sk = _proj(x_skill, Wp_skill.T, bp_skill[None, :])

  cnt_req = _cnt128(req_d2d)
  cnt_rev = _cnt128(rev_d2d)

  agg_s1 = _seg128(h_occ, req_s2d, req_d2d)
  agg_o1 = _seg128(h_sk, rev_s2d, rev_d2d)

  o1, r_o = _conv1(agg_o1, cnt_rev, h_occ, Wl1_so.T, bl1_so[None, :],
                   Wr1_so.T, Wr2_so.T)
  s1, r_s = _conv1(agg_s1, cnt_req, h_sk, Wl1_rs.T, bl1_rs[None, :],
                   Wr1_rs.T, Wr2_rs.T)

  agg_s2 = _seg128(o1, req_s2d, req_d2d)
  agg_o2 = _seg128(s1, rev_s2d, rev_d2d)

  out_occ = _final(agg_o2, cnt_rev, r_o, Wl2_so.T, bl2_so[None, :])
  out_sk = _final(agg_s2, cnt_req, r_s, Wl2_rs.T, bl2_rs[None, :])
  return (out_occ, out_sk)
